# baseline (speedup 1.0000x reference)
import functools

import jax
import jax.numpy as jnp
from jax import lax
from jax.experimental import pallas as pl
from jax.experimental.pallas import tpu as pltpu
from jax.experimental.pallas import tpu_sc as plsc

EMBED_DIM = 300
DP = 304
BATCH = 4096
SEQ = 200
NSPLIT = 2
BATCH_H = BATCH // NSPLIT
TOTAL = BATCH_H * SEQ
NUM_WORKERS = 32
PER_WORKER = TOTAL // NUM_WORKERS
CH = 50
CPS = SEQ // CH
NCHUNKS = PER_WORKER // CH
HALF = NCHUNKS // 2

_mesh = plsc.VectorSubcoreMesh(core_axis_name="c", subcore_axis_name="s")


@functools.partial(
    pl.kernel,
    out_type=jax.ShapeDtypeStruct((BATCH_H, SEQ, EMBED_DIM), jnp.float32),
    mesh=_mesh,
    scratch_types=[
        pltpu.VMEM((NCHUNKS, CH), jnp.int32),
        pltpu.VMEM((CH, DP), jnp.float32),
        pltpu.VMEM((CH, DP), jnp.float32),
        pltpu.VMEM((CH, EMBED_DIM), jnp.float32),
        pltpu.VMEM((CH, EMBED_DIM), jnp.float32),
        pltpu.SemaphoreType.DMA,
        pltpu.SemaphoreType.DMA,
        pltpu.SemaphoreType.DMA,
        pltpu.SemaphoreType.DMA,
    ],
    compiler_params=pltpu.CompilerParams(use_tc_tiling_on_sc=False),
)
def _embedding_gather(idx_hbm, table_hbm, out_hbm,
                      idx_v, buf0, buf1, rows0, rows1,
                      sg0, sg1, sw0, sw1):
    wid = lax.axis_index("s") * 2 + lax.axis_index("c")
    wrow = wid * NCHUNKS
    wsample = wid * (PER_WORKER // SEQ)

    pltpu.sync_copy(idx_hbm.at[pl.ds(wrow, NCHUNKS)], idx_v)

    buf = (buf0, buf1)
    rows = (rows0, rows1)
    sg = (sg0, sg1)
    sw = (sw0, sw1)

    def start(s, j):
        pltpu.async_copy(table_hbm.at[idx_v.at[j]], buf[s], sg[s])

    def finish(s, j):
        pltpu.make_async_copy(table_hbm.at[idx_v.at[j]], buf[s], sg[s]).wait()

        def repack(r, carry):
            src = buf[s].at[r]
            dstr = rows[s].at[r]
            for c in range(0, EMBED_DIM - 16, 16):
                dstr[pl.ds(c, 16)] = src[pl.ds(c, 16)]
            dstr[pl.ds(EMBED_DIM - 16, 16)] = src[pl.ds(EMBED_DIM - 16, 16)]
            return carry

        lax.fori_loop(0, CH, repack, 0)
        bb = wsample + j // CPS
        t0 = (j % CPS) * CH
        pltpu.async_copy(rows[s], out_hbm.at[bb, pl.ds(t0, CH)], sw[s]).wait()

    start(0, 0)

    def body(t, carry):
        start(1, 2 * t + 1)
        finish(0, 2 * t)
        start(0, 2 * t + 2)
        finish(1, 2 * t + 1)
        return carry

    lax.fori_loop(0, HALF - 1, body, 0)

    t_last = HALF - 1
    start(1, 2 * t_last + 1)
    finish(0, 2 * t_last)
    finish(1, 2 * t_last + 1)


def kernel(indices, table):
    flat = indices.reshape(-1).astype(jnp.int32).reshape(NSPLIT * TOTAL // CH, CH)
    table_p = jnp.pad(table, ((0, 0), (0, DP - EMBED_DIM)))
    rows_per_half = TOTAL // CH
    halves = [
        _embedding_gather(flat[h * rows_per_half:(h + 1) * rows_per_half], table_p)
        for h in range(NSPLIT)
    ]
    return jnp.concatenate(halves, axis=0)


# speedup vs baseline: 1.1808x; 1.1808x over previous
import functools

import jax
import jax.numpy as jnp
from jax import lax
from jax.experimental import pallas as pl
from jax.experimental.pallas import tpu as pltpu
from jax.experimental.pallas import tpu_sc as plsc

EMBED_DIM = 300
DP = 304
BATCH = 4096
SEQ = 200
TOTAL = BATCH * SEQ
NUM_WORKERS = 32
PER_WORKER = TOTAL // NUM_WORKERS
CH = 50
CPS = SEQ // CH
NCHUNKS = PER_WORKER // CH
HALF = NCHUNKS // 2

_mesh = plsc.VectorSubcoreMesh(core_axis_name="c", subcore_axis_name="s")


@functools.partial(
    pl.kernel,
    out_type=jax.ShapeDtypeStruct((BATCH, SEQ, EMBED_DIM), jnp.float32),
    mesh=_mesh,
    scratch_types=[
        pltpu.VMEM((NCHUNKS, CH), jnp.int32),
        pltpu.VMEM((CH, DP), jnp.float32),
        pltpu.VMEM((CH, DP), jnp.float32),
        pltpu.VMEM((CH, EMBED_DIM), jnp.float32),
        pltpu.VMEM((CH, EMBED_DIM), jnp.float32),
        pltpu.SemaphoreType.DMA,
        pltpu.SemaphoreType.DMA,
        pltpu.SemaphoreType.DMA,
        pltpu.SemaphoreType.DMA,
    ],
    compiler_params=pltpu.CompilerParams(use_tc_tiling_on_sc=False),
)
def _embedding_gather(idx_hbm, table_hbm, out_hbm,
                      idx_v, buf0, buf1, rows0, rows1,
                      sg0, sg1, sw0, sw1):
    wid = lax.axis_index("s") * 2 + lax.axis_index("c")
    wrow = wid * NCHUNKS
    wsample = wid * (PER_WORKER // SEQ)

    pltpu.sync_copy(idx_hbm.at[pl.ds(wrow, NCHUNKS)], idx_v)

    buf = (buf0, buf1)
    rows = (rows0, rows1)
    sg = (sg0, sg1)
    sw = (sw0, sw1)

    def start(s, j):
        pltpu.async_copy(table_hbm.at[idx_v.at[j]], buf[s], sg[s])

    def finish(s, j):
        pltpu.make_async_copy(table_hbm.at[idx_v.at[j]], buf[s], sg[s]).wait()

        def repack(r, carry):
            src = buf[s].at[r]
            dstr = rows[s].at[r]
            for c in range(0, EMBED_DIM - 16, 16):
                dstr[pl.ds(c, 16)] = src[pl.ds(c, 16)]
            dstr[pl.ds(EMBED_DIM - 16, 16)] = src[pl.ds(EMBED_DIM - 16, 16)]
            return carry

        lax.fori_loop(0, CH, repack, 0)
        bb = wsample + j // CPS
        t0 = (j % CPS) * CH
        pltpu.async_copy(rows[s], out_hbm.at[bb, pl.ds(t0, CH)], sw[s]).wait()

    start(0, 0)

    def body(t, carry):
        start(1, 2 * t + 1)
        finish(0, 2 * t)
        start(0, 2 * t + 2)
        finish(1, 2 * t + 1)
        return carry

    lax.fori_loop(0, HALF - 1, body, 0)

    t_last = HALF - 1
    start(1, 2 * t_last + 1)
    finish(0, 2 * t_last)
    finish(1, 2 * t_last + 1)


def kernel(indices, table):
    flat = indices.reshape(-1).astype(jnp.int32).reshape(TOTAL // CH, CH)
    table_p = jnp.pad(table, ((0, 0), (0, DP - EMBED_DIM)))
    return _embedding_gather(flat, table_p)
